# baseline (speedup 1.0000x reference)
import functools

import jax
import jax.numpy as jnp
from jax import lax
from jax.experimental import pallas as pl
from jax.experimental.pallas import tpu as pltpu
from jax.experimental.pallas import tpu_sc as plsc

M = 320000
K = 8
H = 128
NI = 128
DEPTH = 3

_NC = 2
_NS = 16
NW = _NC * _NS
U = 512
PARTS = [65, 165, 165, 165, 65]
P = len(PARTS)
START = [U * sum(PARTS[:p]) for p in range(P)]
MPS = [U * n for n in PARTS]
C = 16
CH = C * K
GJ = 64
IB = GJ * CH
_RPW_LAST = MPS[-1] // NW
PAD = -(-(_RPW_LAST // C) // GJ) * IB - _RPW_LAST * K


@functools.cache
def _make_sc_gather(part):
    MP = MPS[part]
    RPW = MP // NW
    NCH = RPW // C
    NG = -(-NCH // GJ)
    mesh = plsc.VectorSubcoreMesh(core_axis_name="c", subcore_axis_name="s")

    @functools.partial(
        pl.kernel,
        mesh=mesh,
        out_type=jax.ShapeDtypeStruct((MP * K, H), jnp.float32),
        scratch_types=[
            pltpu.VMEM((IB,), jnp.int32),
            pltpu.VMEM((IB,), jnp.int32),
            pltpu.VMEM((CH, H), jnp.float32),
            pltpu.VMEM((CH, H), jnp.float32),
            pltpu.VMEM((CH, H), jnp.float32),
            pltpu.VMEM((CH, H), jnp.float32),
            pltpu.SemaphoreType.DMA,
            pltpu.SemaphoreType.DMA,
            pltpu.SemaphoreType.DMA,
            pltpu.SemaphoreType.DMA,
            pltpu.SemaphoreType.DMA,
            pltpu.SemaphoreType.DMA,
            pltpu.SemaphoreType.DMA,
            pltpu.SemaphoreType.DMA,
            pltpu.SemaphoreType.DMA,
        ],
    )
    def body(h_hbm, idx_hbm, out_hbm, ibuf0, ibuf1, buf0, buf1, buf2, buf3,
             gs0, gs1, gs2, gs3, ss0, ss1, ss2, ss3, isem):
        wid = lax.axis_index("s") * _NC + lax.axis_index("c")
        base = wid * RPW * K
        gbase = START[part] * K + base
        ibufs = (ibuf0, ibuf1)
        bufs = (buf0, buf1, buf2, buf3)
        gsems = (gs0, gs1, gs2, gs3)
        ssems = (ss0, ss1, ss2, ss3)

        def fire(slot, ibuf, off):
            pltpu.async_copy(
                h_hbm.at[ibuf.at[pl.ds(off, CH)]], bufs[slot], gsems[slot])

        def gwait(slot, ibuf, off):
            pltpu.make_async_copy(
                h_hbm.at[ibuf.at[pl.ds(off, CH)]], bufs[slot],
                gsems[slot]).wait()

        pltpu.sync_copy(idx_hbm.at[pl.ds(gbase, IB)], ibuf0)
        fire(0, ibuf0, 0)
        fire(1, ibuf0, CH)
        fire(2, ibuf0, 2 * CH)

        def group(g, carry):
            for par in range(2):

                @pl.when(lax.rem(g, 2) == par)
                def _():
                    cur = ibufs[par]
                    nxt = ibufs[1 - par]
                    for j in range(GJ):
                        c = g * GJ + j

                        @pl.when(c < NCH)
                        def _():
                            if j == 0:
                                @pl.when(g + 1 < NG)
                                def _():
                                    pltpu.async_copy(
                                        idx_hbm.at[
                                            pl.ds(gbase + (g + 1) * IB, IB)],
                                        nxt, isem)
                            gwait(j % 4, cur, j * CH)
                            if j == GJ - 3:
                                @pl.when(g + 1 < NG)
                                def _():
                                    pltpu.make_async_copy(
                                        idx_hbm.at[
                                            pl.ds(gbase + (g + 1) * IB, IB)],
                                        nxt, isem).wait()

                            @pl.when(c + 3 < NCH)
                            def _():
                                s3 = (j + 3) % 4
                                @pl.when(c >= 1)
                                def _():
                                    pltpu.make_async_copy(
                                        bufs[s3],
                                        out_hbm.at[
                                            pl.ds(base + (c - 1) * CH, CH)],
                                        ssems[s3]).wait()
                                if j < GJ - 3:
                                    fire(s3, cur, (j + 3) * CH)
                                else:
                                    fire(s3, nxt, (j + 3 - GJ) * CH)

                            pltpu.async_copy(
                                bufs[j % 4],
                                out_hbm.at[pl.ds(base + c * CH, CH)],
                                ssems[j % 4])
            return carry

        lax.fori_loop(0, NG, group, 0)

        for c in (NCH - 4, NCH - 3, NCH - 2, NCH - 1):
            pltpu.make_async_copy(
                bufs[c % 4], out_hbm.at[pl.ds(base + c * CH, CH)],
                ssems[c % 4]).wait()

    return body


BMP = 1280


def _pre_body(x_ref, w_ref, b_ref, out_ref):
    out_ref[...] = (
        jnp.dot(x_ref[...], w_ref[...], preferred_element_type=jnp.float32)
        + b_ref[...]
    ).astype(jnp.bfloat16)


_pre_call = pl.pallas_call(
    _pre_body,
    grid=(M // BMP,),
    in_specs=[
        pl.BlockSpec((BMP, NI), lambda i: (i, 0)),
        pl.BlockSpec((NI, 3 * H), lambda i: (0, 0)),
        pl.BlockSpec((1, 3 * H), lambda i: (0, 0)),
    ],
    out_specs=pl.BlockSpec((BMP, 3 * H), lambda i: (i, 0)),
    out_shape=jax.ShapeDtypeStruct((M, 3 * H), jnp.bfloat16),
)


BM = 640


def _upd_body(nei_ref, pre_ref, wz2_ref, wh2_ref, ur_ref, *rest, mask_row0):
    out_ref = rest[-1]
    nei3 = nei_ref[...]
    nei2 = nei3.reshape(K * BM, H)
    r2 = jnp.dot(nei2, ur_ref[...], preferred_element_type=jnp.float32)
    pre = pre_ref[...].astype(jnp.float32)
    r1 = pre[:, 0:H][None, :, :]
    r = 0.5 + 0.5 * jnp.tanh(0.5 * (r1 + r2.reshape(K, BM, H)))
    sum_h = jnp.sum(nei3, axis=0)
    sum_g = jnp.sum(r * nei3, axis=0)
    z = 0.5 + 0.5 * jnp.tanh(0.5 * (
        pre[:, H:2 * H]
        + jnp.dot(sum_h, wz2_ref[...], preferred_element_type=jnp.float32)))
    ph = jnp.tanh(
        pre[:, 2 * H:3 * H]
        + jnp.dot(sum_g, wh2_ref[...], preferred_element_type=jnp.float32))
    hn = (1.0 - z) * sum_h + z * ph
    if mask_row0:
        rid = (pl.program_id(0) * BM
               + lax.broadcasted_iota(jnp.int32, (BM, 1), 0))
        hn = jnp.where(rid == 0, 0.0, hn)
    out_ref[...] = hn


@functools.cache
def _make_upd(part):
    poff = START[part] // BM
    in_specs = [
        pl.BlockSpec((K, BM, H), lambda i: (0, i, 0)),
        pl.BlockSpec((BM, 3 * H), lambda i: (poff + i, 0)),
        pl.BlockSpec((H, H), lambda i: (0, 0)),
        pl.BlockSpec((H, H), lambda i: (0, 0)),
        pl.BlockSpec((H, H), lambda i: (0, 0)),
    ]
    aliases = {}
    if part > 0:
        in_specs += [pl.BlockSpec(memory_space=pl.ANY)]
        aliases = {5: 0}
    return pl.pallas_call(
        functools.partial(_upd_body, mask_row0=(part == 0)),
        grid=(MPS[part] // BM,),
        in_specs=in_specs,
        out_specs=pl.BlockSpec((BM, H), lambda i: (poff + i, 0)),
        out_shape=jax.ShapeDtypeStruct((M, H), jnp.float32),
        input_output_aliases=aliases,
    )


def kernel(h, x, mess_graph, W_z, b_z, W_r, U_r, b_ur, W_h, b_h):
    mg = mess_graph.astype(jnp.int32)
    idx = jnp.concatenate(
        [mg[START[p]:START[p] + MPS[p]].T.reshape(-1) for p in range(P)])
    idx = jnp.pad(idx, (0, PAD))
    w_cat = jnp.concatenate([W_r, W_z[:NI], W_h[:NI]], axis=1)
    b_cat = jnp.concatenate([b_ur, b_z, b_h]).reshape(1, 3 * H)
    pre = _pre_call(x, w_cat, b_cat)
    wz2 = W_z[NI:]
    wh2 = W_h[NI:]
    for _ in range(DEPTH):
        neis = [_make_sc_gather(p)(h, idx) for p in range(P)]
        hf = _make_upd(0)(neis[0].reshape(K, MPS[0], H), pre, wz2, wh2, U_r)
        for p in range(1, P):
            hf = _make_upd(p)(neis[p].reshape(K, MPS[p], H), pre, wz2, wh2,
                              U_r, hf)
        h = hf
    return h


# speedup vs baseline: 1.0339x; 1.0339x over previous
import functools

import jax
import jax.numpy as jnp
from jax import lax
from jax.experimental import pallas as pl
from jax.experimental.pallas import tpu as pltpu
from jax.experimental.pallas import tpu_sc as plsc

M = 320000
K = 8
H = 128
NI = 128
DEPTH = 3

_NC = 2
_NS = 16
NW = _NC * _NS
U = 512
PARTS = [125, 125, 125, 125, 125]
P = len(PARTS)
START = [U * sum(PARTS[:p]) for p in range(P)]
MPS = [U * n for n in PARTS]
C = 16
CH = C * K
GJ = 64
IB = GJ * CH
_RPW_LAST = MPS[-1] // NW
PAD = -(-(_RPW_LAST // C) // GJ) * IB - _RPW_LAST * K


@functools.cache
def _make_sc_gather(part):
    MP = MPS[part]
    RPW = MP // NW
    NCH = RPW // C
    NG = -(-NCH // GJ)
    mesh = plsc.VectorSubcoreMesh(core_axis_name="c", subcore_axis_name="s")

    @functools.partial(
        pl.kernel,
        mesh=mesh,
        out_type=jax.ShapeDtypeStruct((MP * K, H), jnp.float32),
        scratch_types=[
            pltpu.VMEM((IB,), jnp.int32),
            pltpu.VMEM((IB,), jnp.int32),
            pltpu.VMEM((CH, H), jnp.float32),
            pltpu.VMEM((CH, H), jnp.float32),
            pltpu.VMEM((CH, H), jnp.float32),
            pltpu.VMEM((CH, H), jnp.float32),
            pltpu.SemaphoreType.DMA,
            pltpu.SemaphoreType.DMA,
            pltpu.SemaphoreType.DMA,
            pltpu.SemaphoreType.DMA,
            pltpu.SemaphoreType.DMA,
            pltpu.SemaphoreType.DMA,
            pltpu.SemaphoreType.DMA,
            pltpu.SemaphoreType.DMA,
            pltpu.SemaphoreType.DMA,
        ],
    )
    def body(h_hbm, idx_hbm, out_hbm, ibuf0, ibuf1, buf0, buf1, buf2, buf3,
             gs0, gs1, gs2, gs3, ss0, ss1, ss2, ss3, isem):
        wid = lax.axis_index("s") * _NC + lax.axis_index("c")
        base = wid * RPW * K
        gbase = START[part] * K + base
        ibufs = (ibuf0, ibuf1)
        bufs = (buf0, buf1, buf2, buf3)
        gsems = (gs0, gs1, gs2, gs3)
        ssems = (ss0, ss1, ss2, ss3)

        def fire(slot, ibuf, off):
            pltpu.async_copy(
                h_hbm.at[ibuf.at[pl.ds(off, CH)]], bufs[slot], gsems[slot])

        def gwait(slot, ibuf, off):
            pltpu.make_async_copy(
                h_hbm.at[ibuf.at[pl.ds(off, CH)]], bufs[slot],
                gsems[slot]).wait()

        pltpu.sync_copy(idx_hbm.at[pl.ds(gbase, IB)], ibuf0)
        fire(0, ibuf0, 0)
        fire(1, ibuf0, CH)
        fire(2, ibuf0, 2 * CH)

        def group(g, carry):
            for par in range(2):

                @pl.when(lax.rem(g, 2) == par)
                def _():
                    cur = ibufs[par]
                    nxt = ibufs[1 - par]
                    for j in range(GJ):
                        c = g * GJ + j

                        @pl.when(c < NCH)
                        def _():
                            if j == 0:
                                @pl.when(g + 1 < NG)
                                def _():
                                    pltpu.async_copy(
                                        idx_hbm.at[
                                            pl.ds(gbase + (g + 1) * IB, IB)],
                                        nxt, isem)
                            gwait(j % 4, cur, j * CH)
                            if j == GJ - 3:
                                @pl.when(g + 1 < NG)
                                def _():
                                    pltpu.make_async_copy(
                                        idx_hbm.at[
                                            pl.ds(gbase + (g + 1) * IB, IB)],
                                        nxt, isem).wait()

                            @pl.when(c + 3 < NCH)
                            def _():
                                s3 = (j + 3) % 4
                                @pl.when(c >= 1)
                                def _():
                                    pltpu.make_async_copy(
                                        bufs[s3],
                                        out_hbm.at[
                                            pl.ds(base + (c - 1) * CH, CH)],
                                        ssems[s3]).wait()
                                if j < GJ - 3:
                                    fire(s3, cur, (j + 3) * CH)
                                else:
                                    fire(s3, nxt, (j + 3 - GJ) * CH)

                            pltpu.async_copy(
                                bufs[j % 4],
                                out_hbm.at[pl.ds(base + c * CH, CH)],
                                ssems[j % 4])
            return carry

        lax.fori_loop(0, NG, group, 0)

        for c in (NCH - 4, NCH - 3, NCH - 2, NCH - 1):
            pltpu.make_async_copy(
                bufs[c % 4], out_hbm.at[pl.ds(base + c * CH, CH)],
                ssems[c % 4]).wait()

    return body


BMP = 1280


def _pre_body(x_ref, w_ref, b_ref, out_ref):
    out_ref[...] = (
        jnp.dot(x_ref[...], w_ref[...], preferred_element_type=jnp.float32)
        + b_ref[...]
    ).astype(jnp.bfloat16)


_pre_call = pl.pallas_call(
    _pre_body,
    grid=(M // BMP,),
    in_specs=[
        pl.BlockSpec((BMP, NI), lambda i: (i, 0)),
        pl.BlockSpec((NI, 3 * H), lambda i: (0, 0)),
        pl.BlockSpec((1, 3 * H), lambda i: (0, 0)),
    ],
    out_specs=pl.BlockSpec((BMP, 3 * H), lambda i: (i, 0)),
    out_shape=jax.ShapeDtypeStruct((M, 3 * H), jnp.bfloat16),
)


BM = 800


def _upd_body(nei_ref, pre_ref, wz2_ref, wh2_ref, ur_ref, *rest, mask_row0):
    out_ref = rest[-1]
    nei3 = nei_ref[...]
    nei2 = nei3.reshape(K * BM, H)
    r2 = jnp.dot(nei2, ur_ref[...], preferred_element_type=jnp.float32)
    pre = pre_ref[...].astype(jnp.float32)
    r1 = pre[:, 0:H][None, :, :]
    r = 0.5 + 0.5 * jnp.tanh(0.5 * (r1 + r2.reshape(K, BM, H)))
    sum_h = jnp.sum(nei3, axis=0)
    sum_g = jnp.sum(r * nei3, axis=0)
    z = 0.5 + 0.5 * jnp.tanh(0.5 * (
        pre[:, H:2 * H]
        + jnp.dot(sum_h, wz2_ref[...], preferred_element_type=jnp.float32)))
    ph = jnp.tanh(
        pre[:, 2 * H:3 * H]
        + jnp.dot(sum_g, wh2_ref[...], preferred_element_type=jnp.float32))
    hn = (1.0 - z) * sum_h + z * ph
    if mask_row0:
        rid = (pl.program_id(0) * BM
               + lax.broadcasted_iota(jnp.int32, (BM, 1), 0))
        hn = jnp.where(rid == 0, 0.0, hn)
    out_ref[...] = hn


@functools.cache
def _make_upd(part):
    poff = START[part] // BM
    in_specs = [
        pl.BlockSpec((K, BM, H), lambda i: (0, i, 0)),
        pl.BlockSpec((BM, 3 * H), lambda i: (poff + i, 0)),
        pl.BlockSpec((H, H), lambda i: (0, 0)),
        pl.BlockSpec((H, H), lambda i: (0, 0)),
        pl.BlockSpec((H, H), lambda i: (0, 0)),
    ]
    aliases = {}
    if part > 0:
        in_specs += [pl.BlockSpec(memory_space=pl.ANY)]
        aliases = {5: 0}
    return pl.pallas_call(
        functools.partial(_upd_body, mask_row0=(part == 0)),
        grid=(MPS[part] // BM,),
        in_specs=in_specs,
        out_specs=pl.BlockSpec((BM, H), lambda i: (poff + i, 0)),
        out_shape=jax.ShapeDtypeStruct((M, H), jnp.float32),
        input_output_aliases=aliases,
    )


def kernel(h, x, mess_graph, W_z, b_z, W_r, U_r, b_ur, W_h, b_h):
    mg = mess_graph.astype(jnp.int32)
    idx = jnp.concatenate(
        [mg[START[p]:START[p] + MPS[p]].T.reshape(-1) for p in range(P)])
    idx = jnp.pad(idx, (0, PAD))
    w_cat = jnp.concatenate([W_r, W_z[:NI], W_h[:NI]], axis=1)
    b_cat = jnp.concatenate([b_ur, b_z, b_h]).reshape(1, 3 * H)
    pre = _pre_call(x, w_cat, b_cat)
    wz2 = W_z[NI:]
    wh2 = W_h[NI:]
    for _ in range(DEPTH):
        neis = [_make_sc_gather(p)(h, idx) for p in range(P)]
        hf = _make_upd(0)(neis[0].reshape(K, MPS[0], H), pre, wz2, wh2, U_r)
        for p in range(1, P):
            hf = _make_upd(p)(neis[p].reshape(K, MPS[p], H), pre, wz2, wh2,
                              U_r, hf)
        h = hf
    return h
